# baseline (device time: 14511 ns/iter reference)
import os

import jax
import jax.numpy as jnp
from jax import lax
from jax.experimental import pallas as pl
from jax.experimental.pallas import tpu as pltpu

_NO_COMM = os.environ.get("KERNEL_NO_COMM") == "1"
_NO_COMPUTE = os.environ.get("KERNEL_NO_COMPUTE") == "1"

N_ROWS = 1024
N_COLS = 512
CHUNK = 128
MAX_CHUNKS = N_ROWS // CHUNK


def kernel(x, dest):
    dest2d = dest.reshape(1, N_ROWS)

    def body(x_ref, dest_ref, out_ref, send_buf, recv_buf, send_sems, recv_sems):
        my_x = lax.axis_index("x")
        my_y = lax.axis_index("y")
        my_z = lax.axis_index("z")
        partner = (1 - my_x, my_y, my_z)

        if not _NO_COMM:
            bsem = pltpu.get_barrier_semaphore()
            pl.semaphore_signal(
                bsem, inc=1, device_id=partner, device_id_type=pl.DeviceIdType.MESH
            )

        d = dest_ref[:, :]
        sm = (d != my_x).astype(jnp.float32)

        i_row = lax.broadcasted_iota(jnp.int32, (N_ROWS, N_ROWS), 0)
        j_row = lax.broadcasted_iota(jnp.int32, (N_ROWS, N_ROWS), 1)
        upper = (i_row < j_row).astype(jnp.bfloat16)
        scum = jnp.dot(
            sm.astype(jnp.bfloat16), upper, preferred_element_type=jnp.float32
        )
        pos = lax.broadcasted_iota(jnp.int32, (1, N_ROWS), 1)
        scum_i = scum.astype(jnp.int32)

        s_count = jnp.sum(sm).astype(jnp.int32)
        k_count = N_ROWS - s_count

        pos_v = jnp.where(sm > 0, scum_i, s_count + (pos - scum_i))

        xb = x_ref[:, :].astype(jnp.bfloat16)
        nch = (s_count + CHUNK - 1) // CHUNK
        if _NO_COMPUTE:
            nch = 4

        if not _NO_COMM:
            pl.semaphore_wait(bsem, 1)

        def chunk_rdma(j):
            return pltpu.make_async_remote_copy(
                src_ref=send_buf.at[pl.ds(j * CHUNK, CHUNK), :],
                dst_ref=recv_buf.at[pl.ds(j * CHUNK, CHUNK), :],
                send_sem=send_sems.at[j],
                recv_sem=recv_sems.at[j],
                device_id=partner,
                device_id_type=pl.DeviceIdType.MESH,
            )

        ii_c = lax.broadcasted_iota(jnp.int32, (CHUNK, N_ROWS), 0)
        for j in range(MAX_CHUNKS):
            if _NO_COMPUTE:
                send_buf[pl.ds(j * CHUNK, CHUNK), :] = x_ref[
                    pl.ds(j * CHUNK, CHUNK), :
                ].astype(jnp.bfloat16)
            else:
                p_j = (ii_c == pos_v - j * CHUNK).astype(jnp.bfloat16)
                send_buf[pl.ds(j * CHUNK, CHUNK), :] = jnp.dot(
                    p_j, xb, preferred_element_type=jnp.float32
                ).astype(jnp.bfloat16)
            if not _NO_COMM:
                @pl.when(j < nch)
                def _(j=j):
                    chunk_rdma(j).start()

        if not _NO_COMM:
            for j in range(MAX_CHUNKS):
                @pl.when(j < nch)
                def _(j=j):
                    chunk_rdma(j).wait_recv()

        row1 = lax.broadcasted_iota(jnp.int32, (N_ROWS, 1), 0)
        comb = jnp.where(row1 < s_count, recv_buf[:, :], send_buf[:, :])
        shift = jnp.where(my_x == 0, k_count, 0)
        out_ref[:, :] = pltpu.roll(comb, shift, 0)

        if not _NO_COMM:
            for j in range(MAX_CHUNKS):
                @pl.when(j < nch)
                def _(j=j):
                    chunk_rdma(j).wait_send()

    return pl.pallas_call(
        body,
        out_shape=jax.ShapeDtypeStruct((N_ROWS, N_COLS), jnp.bfloat16),
        in_specs=[
            pl.BlockSpec(memory_space=pltpu.VMEM),
            pl.BlockSpec(memory_space=pltpu.VMEM),
        ],
        out_specs=pl.BlockSpec(memory_space=pltpu.VMEM),
        scratch_shapes=[
            pltpu.VMEM((N_ROWS, N_COLS), jnp.bfloat16),
            pltpu.VMEM((N_ROWS, N_COLS), jnp.bfloat16),
            pltpu.SemaphoreType.DMA((MAX_CHUNKS,)),
            pltpu.SemaphoreType.DMA((MAX_CHUNKS,)),
        ],
        compiler_params=pltpu.CompilerParams(
            collective_id=None if _NO_COMM else 0
        ),
    )(x, dest2d)


# device time: 13818 ns/iter; 1.0502x vs baseline; 1.0502x over previous
import os

import jax
import jax.numpy as jnp
from jax import lax
from jax.experimental import pallas as pl
from jax.experimental.pallas import tpu as pltpu

_NO_COMM = os.environ.get("KERNEL_NO_COMM") == "1"
_NO_COMPUTE = os.environ.get("KERNEL_NO_COMPUTE") == "1"
_NO_RDMA = os.environ.get("KERNEL_NO_RDMA") == "1"

N_ROWS = 1024
N_COLS = 512
CHUNK = int(os.environ.get("KERNEL_CHUNK", "128"))
MAX_CHUNKS = N_ROWS // CHUNK


def kernel(x, dest):
    dest2d = dest.reshape(1, N_ROWS)

    def body(x_ref, dest_ref, out_ref, send_buf, recv_buf, send_sems, recv_sems):
        my_x = lax.axis_index("x")
        my_y = lax.axis_index("y")
        my_z = lax.axis_index("z")
        partner = (1 - my_x, my_y, my_z)

        if not _NO_COMM:
            bsem = pltpu.get_barrier_semaphore()
            pl.semaphore_signal(
                bsem, inc=1, device_id=partner, device_id_type=pl.DeviceIdType.MESH
            )

        d = dest_ref[:, :]
        sm = (d != my_x).astype(jnp.bfloat16)

        i_row = lax.broadcasted_iota(jnp.int32, (N_ROWS, N_ROWS), 0)
        j_row = lax.broadcasted_iota(jnp.int32, (N_ROWS, N_ROWS), 1)
        upper = (i_row < j_row).astype(jnp.bfloat16)
        scum_i = jnp.dot(
            sm, upper, preferred_element_type=jnp.float32
        ).astype(jnp.int32)
        pos = lax.broadcasted_iota(jnp.int32, (1, N_ROWS), 1)

        s_count = jnp.sum(sm, dtype=jnp.float32).astype(jnp.int32)
        k_count = N_ROWS - s_count

        pos_v = jnp.where(sm > 0, scum_i, s_count + (pos - scum_i))
        off_recv = jnp.where(my_x == 0, k_count, 0)

        xb = x_ref[:, :].astype(jnp.bfloat16)
        nch = (s_count + CHUNK - 1) // CHUNK
        if _NO_COMPUTE:
            nch = 512 // CHUNK

        if not _NO_COMM:
            pl.semaphore_wait(bsem, 1)

        def chunk_rdma(j):
            return pltpu.make_async_remote_copy(
                src_ref=send_buf.at[pl.ds(j * CHUNK, CHUNK), :],
                dst_ref=recv_buf.at[pl.ds(j * CHUNK, CHUNK), :],
                send_sem=send_sems.at[j],
                recv_sem=recv_sems.at[j],
                device_id=partner,
                device_id_type=pl.DeviceIdType.MESH,
            )

        ii_c = lax.broadcasted_iota(jnp.int32, (CHUNK, N_ROWS), 0)
        for j in range(MAX_CHUNKS):
            if _NO_COMPUTE:
                send_buf[pl.ds(j * CHUNK, CHUNK), :] = x_ref[
                    pl.ds(j * CHUNK, CHUNK), :
                ].astype(jnp.bfloat16)
            else:
                p_j = (ii_c == pos_v - j * CHUNK).astype(jnp.bfloat16)
                send_buf[pl.ds(j * CHUNK, CHUNK), :] = jnp.dot(
                    p_j, xb, preferred_element_type=jnp.float32
                ).astype(jnp.bfloat16)
            if not (_NO_COMM or _NO_RDMA):
                @pl.when(j < nch)
                def _(j=j):
                    chunk_rdma(j).start()

        if _NO_COMPUTE and not (_NO_COMM or _NO_RDMA):
            for j in range(MAX_CHUNKS):
                @pl.when(j < nch)
                def _(j=j):
                    chunk_rdma(j).wait_recv()

        if _NO_COMPUTE:
            out_ref[:, :] = send_buf[:, :]
        else:
            row1 = lax.broadcasted_iota(jnp.int32, (N_ROWS, 1), 0)
            rolled = pltpu.roll(send_buf[:, :], off_recv, 0)
            in_recv = (row1 >= off_recv) & (row1 < off_recv + s_count)
            out_ref[:, :] = jnp.where(
                in_recv, jnp.bfloat16(0), rolled
            )

        if not (_NO_COMM or _NO_COMPUTE):
            jj_c = lax.broadcasted_iota(jnp.int32, (N_ROWS, CHUNK), 1)
            i_col = lax.broadcasted_iota(jnp.int32, (N_ROWS, CHUNK), 0)
            for j in range(MAX_CHUNKS):
                @pl.when(j < nch)
                def _(j=j):
                    chunk_rdma(j).wait_recv()
                    pr_j = (
                        (i_col == off_recv + j * CHUNK + jj_c)
                        & (j * CHUNK + jj_c < s_count)
                    ).astype(jnp.bfloat16)
                    out_ref[:, :] = out_ref[:, :] + jnp.dot(
                        pr_j,
                        recv_buf[pl.ds(j * CHUNK, CHUNK), :],
                        preferred_element_type=jnp.float32,
                    ).astype(jnp.bfloat16)

        if not (_NO_COMM or _NO_RDMA):
            for j in range(MAX_CHUNKS):
                @pl.when(j < nch)
                def _(j=j):
                    chunk_rdma(j).wait_send()

    return pl.pallas_call(
        body,
        out_shape=jax.ShapeDtypeStruct((N_ROWS, N_COLS), jnp.bfloat16),
        in_specs=[
            pl.BlockSpec(memory_space=pltpu.VMEM),
            pl.BlockSpec(memory_space=pltpu.VMEM),
        ],
        out_specs=pl.BlockSpec(memory_space=pltpu.VMEM),
        scratch_shapes=[
            pltpu.VMEM((N_ROWS, N_COLS), jnp.bfloat16),
            pltpu.VMEM((N_ROWS, N_COLS), jnp.bfloat16),
            pltpu.SemaphoreType.DMA((MAX_CHUNKS,)),
            pltpu.SemaphoreType.DMA((MAX_CHUNKS,)),
        ],
        compiler_params=pltpu.CompilerParams(
            collective_id=None if _NO_COMM else 0
        ),
    )(x, dest2d)
